# Initial kernel scaffold; baseline (speedup 1.0000x reference)
import jax
import jax.numpy as jnp
from jax.experimental import pallas as pl


def kernel(pos, atomic_number, edge_index, W_in, b_in, W1_l, b1, W1_r, W2_l, b2, W2_r, W_out, b_out):
    raise NotImplementedError("write your pallas kernel here")


# speedup vs baseline: 3.2127x; 3.2127x over previous
import jax
import jax.numpy as jnp
import numpy as np
from jax import lax
from jax.experimental import pallas as pl
from jax.experimental.pallas import tpu as pltpu
from jax.experimental.pallas import tpu_sc as plsc

N = 10000
E = 320000
HID = 128

NC = 2
NS = 16
NW = NC * NS
CHUNK = 64
CPW = 160
EPW = CPW * CHUNK
EP = NW * EPW
NPAD = 10240
RPT = NPAD // NS
CNTW = 16
BN = 1024


def _sc_agg_body(h_hbm, srcw_hbm, dstw_hbm, out_hbm,
                 src_v, dst_v, buf0, buf1, acc_sh, sem0, sem1):
    cid = lax.axis_index("c")
    sid = lax.axis_index("s")
    wid = sid * NC + cid

    pltpu.sync_copy(srcw_hbm.at[wid], src_v)
    pltpu.sync_copy(dstw_hbm.at[wid], dst_v)

    zvec = jnp.zeros((16,), jnp.float32)

    def _zero_row(i, _):
        for j in range(HID // 16):
            buf0[i, pl.ds(j * 16, 16)] = zvec
        return 0

    lax.fori_loop(0, CHUNK, _zero_row, 0)
    for r in range(RPT // CHUNK):
        pltpu.sync_copy(buf0, acc_sh.at[pl.ds(sid * RPT + r * CHUNK, CHUNK)])

    plsc.subcore_barrier()

    def _pair(k, _):
        c0 = 2 * k
        c1 = 2 * k + 1
        cp0 = pltpu.async_copy(
            h_hbm.at[src_v.at[pl.ds(c0 * CHUNK, CHUNK)]], buf0, sem0)
        cp1 = pltpu.async_copy(
            h_hbm.at[src_v.at[pl.ds(c1 * CHUNK, CHUNK)]], buf1, sem1)
        cp0.wait()
        pltpu.sync_copy(buf0, acc_sh.at[dst_v.at[c0]], add=True)
        cp1.wait()
        pltpu.sync_copy(buf1, acc_sh.at[dst_v.at[c1]], add=True)
        return 0

    lax.fori_loop(0, CPW // 2, _pair, 0)

    plsc.subcore_barrier()

    pltpu.sync_copy(acc_sh.at[pl.ds(sid * RPT, RPT)],
                    out_hbm.at[cid, pl.ds(sid * RPT, RPT)])


_sc_agg = pl.kernel(
    _sc_agg_body,
    out_type=[jax.ShapeDtypeStruct((NC, NPAD, HID), jnp.float32)],
    mesh=plsc.VectorSubcoreMesh(core_axis_name="c", subcore_axis_name="s"),
    scratch_types=[
        pltpu.VMEM((EPW,), jnp.int32),
        pltpu.VMEM((CPW, CHUNK), jnp.int32),
        pltpu.VMEM((CHUNK, HID), jnp.float32),
        pltpu.VMEM((CHUNK, HID), jnp.float32),
        pltpu.VMEM_SHARED((NPAD, HID), jnp.float32),
        pltpu.SemaphoreType.DMA,
        pltpu.SemaphoreType.DMA,
    ],
    compiler_params=pltpu.CompilerParams(use_tc_tiling_on_sc=False),
)


def _sc_cnt_body(dstw_hbm, cnt_hbm, dst_v, ones_v, zc_v, cnt_sh):
    cid = lax.axis_index("c")
    sid = lax.axis_index("s")
    wid = sid * NC + cid

    pltpu.sync_copy(dstw_hbm.at[wid], dst_v)

    ovec = jnp.full((16,), 1.0, jnp.float32)
    zvec = jnp.zeros((16,), jnp.float32)

    def _fill(i, _):
        ones_v[i, :] = ovec
        zc_v[i, :] = zvec
        return 0

    lax.fori_loop(0, CHUNK, _fill, 0)
    for r in range(RPT // CHUNK):
        pltpu.sync_copy(zc_v, cnt_sh.at[pl.ds(sid * RPT + r * CHUNK, CHUNK)])

    plsc.subcore_barrier()

    def _chunk(c, _):
        pltpu.sync_copy(ones_v, cnt_sh.at[dst_v.at[c]], add=True)
        return 0

    lax.fori_loop(0, CPW, _chunk, 0)

    plsc.subcore_barrier()

    pltpu.sync_copy(cnt_sh.at[pl.ds(sid * RPT, RPT)],
                    cnt_hbm.at[cid, pl.ds(sid * RPT, RPT)])


_sc_cnt = pl.kernel(
    _sc_cnt_body,
    out_type=[jax.ShapeDtypeStruct((NC, NPAD, CNTW), jnp.float32)],
    mesh=plsc.VectorSubcoreMesh(core_axis_name="c", subcore_axis_name="s"),
    scratch_types=[
        pltpu.VMEM((CPW, CHUNK), jnp.int32),
        pltpu.VMEM((CHUNK, CNTW), jnp.float32),
        pltpu.VMEM((CHUNK, CNTW), jnp.float32),
        pltpu.VMEM_SHARED((NPAD, CNTW), jnp.float32),
    ],
    compiler_params=pltpu.CompilerParams(use_tc_tiling_on_sc=False),
)


def _pre_body(x_ref, w_ref, b_ref, o_ref):
    o_ref[...] = jnp.maximum(
        jnp.dot(x_ref[...], w_ref[...], preferred_element_type=jnp.float32)
        + b_ref[...], 0.0)


def _tc_pre(x, w8, b):
    return pl.pallas_call(
        _pre_body,
        grid=(NPAD // BN,),
        in_specs=[
            pl.BlockSpec((BN, 8), lambda i: (i, 0)),
            pl.BlockSpec((8, HID), lambda i: (0, 0)),
            pl.BlockSpec((1, HID), lambda i: (0, 0)),
        ],
        out_specs=pl.BlockSpec((BN, HID), lambda i: (i, 0)),
        out_shape=jax.ShapeDtypeStruct((NPAD, HID), jnp.float32),
    )(x, w8, b)


def _mid_body(p_ref, c_ref, h_ref, wl_ref, wr_ref, b_ref, o_ref):
    s = p_ref[0] + p_ref[1]
    cnt = c_ref[0, :, 0:1] + c_ref[1, :, 0:1]
    mean = s / jnp.maximum(cnt, 1.0)
    o_ref[...] = jnp.maximum(
        jnp.dot(mean, wl_ref[...], preferred_element_type=jnp.float32)
        + jnp.dot(h_ref[...], wr_ref[...], preferred_element_type=jnp.float32)
        + b_ref[...], 0.0)


def _tc_mid(p, c, h, wlT, wrT, b):
    return pl.pallas_call(
        _mid_body,
        grid=(NPAD // BN,),
        in_specs=[
            pl.BlockSpec((NC, BN, HID), lambda i: (0, i, 0)),
            pl.BlockSpec((NC, BN, CNTW), lambda i: (0, i, 0)),
            pl.BlockSpec((BN, HID), lambda i: (i, 0)),
            pl.BlockSpec((HID, HID), lambda i: (0, 0)),
            pl.BlockSpec((HID, HID), lambda i: (0, 0)),
            pl.BlockSpec((1, HID), lambda i: (0, 0)),
        ],
        out_specs=pl.BlockSpec((BN, HID), lambda i: (i, 0)),
        out_shape=jax.ShapeDtypeStruct((NPAD, HID), jnp.float32),
    )(p, c, h, wlT, wrT, b)


def _fin_body(p_ref, c_ref, h_ref, wl_ref, wr_ref, b_ref, wo_ref, bo_ref,
              o_ref, m_ref):
    i = pl.program_id(0)
    s = p_ref[0] + p_ref[1]
    cnt = c_ref[0, :, 0:1] + c_ref[1, :, 0:1]
    mean = s / jnp.maximum(cnt, 1.0)
    h2 = jnp.maximum(
        jnp.dot(mean, wl_ref[...], preferred_element_type=jnp.float32)
        + jnp.dot(h_ref[...], wr_ref[...], preferred_element_type=jnp.float32)
        + b_ref[...], 0.0)
    ns = (jnp.dot(h2, wo_ref[...], preferred_element_type=jnp.float32)
          + bo_ref[...])
    o_ref[...] = ns
    row = i * BN + lax.broadcasted_iota(jnp.int32, (BN, 1), 0)
    valid = (row < N).astype(jnp.float32)
    part = jnp.sum(ns * valid, axis=0, keepdims=True)

    @pl.when(i == 0)
    def _():
        m_ref[...] = jnp.zeros_like(m_ref)

    acc = m_ref[...] + part
    m_ref[...] = jnp.where(i == NPAD // BN - 1,
                           acc * np.float32(1.0 / N), acc)


def _tc_final(p, c, h, wlT, wrT, b, woT, bo):
    return pl.pallas_call(
        _fin_body,
        grid=(NPAD // BN,),
        in_specs=[
            pl.BlockSpec((NC, BN, HID), lambda i: (0, i, 0)),
            pl.BlockSpec((NC, BN, CNTW), lambda i: (0, i, 0)),
            pl.BlockSpec((BN, HID), lambda i: (i, 0)),
            pl.BlockSpec((HID, HID), lambda i: (0, 0)),
            pl.BlockSpec((HID, HID), lambda i: (0, 0)),
            pl.BlockSpec((1, HID), lambda i: (0, 0)),
            pl.BlockSpec((HID, HID), lambda i: (0, 0)),
            pl.BlockSpec((1, HID), lambda i: (0, 0)),
        ],
        out_specs=[
            pl.BlockSpec((BN, HID), lambda i: (i, 0)),
            pl.BlockSpec((1, HID), lambda i: (0, 0)),
        ],
        out_shape=[
            jax.ShapeDtypeStruct((NPAD, HID), jnp.float32),
            jax.ShapeDtypeStruct((1, HID), jnp.float32),
        ],
    )(p, c, h, wlT, wrT, b, woT, bo)


def kernel(pos, atomic_number, edge_index,
           W_in, b_in, W1_l, b1, W1_r, W2_l, b2, W2_r, W_out, b_out):
    f32 = jnp.float32
    z = atomic_number.astype(f32)[:, None] / 10.0
    x = jnp.concatenate([z, pos, jnp.zeros((N, 4), f32)], axis=1)
    x = jnp.concatenate([x, jnp.zeros((NPAD - N, 8), f32)], axis=0)
    w8 = jnp.concatenate([W_in.T, jnp.zeros((4, HID), f32)], axis=0)

    src = edge_index[0]
    dst = edge_index[1]
    pad = EP - E
    srcw = jnp.concatenate([src, jnp.zeros((pad,), jnp.int32)]).reshape(NW, EPW)
    dstw = jnp.concatenate(
        [dst, jnp.full((pad,), N, jnp.int32)]).reshape(NW, CPW, CHUNK)

    b_in2 = b_in.reshape(1, HID)
    b1_2 = b1.reshape(1, HID)
    b2_2 = b2.reshape(1, HID)
    bo_2 = b_out.reshape(1, HID)

    (cnt,) = _sc_cnt(dstw)
    h0 = _tc_pre(x, w8, b_in2)
    (p1,) = _sc_agg(h0, srcw, dstw)
    h1 = _tc_mid(p1, cnt, h0, W1_l.T, W1_r.T, b1_2)
    (p2,) = _sc_agg(h1, srcw, dstw)
    ns, mvec = _tc_final(p2, cnt, h1, W2_l.T, W2_r.T, b2_2, W_out.T, bo_2)
    return mvec.reshape(HID), ns[:N]


# speedup vs baseline: 6.5035x; 2.0243x over previous
import jax
import jax.numpy as jnp
import numpy as np
from jax import lax
from jax.experimental import pallas as pl
from jax.experimental.pallas import tpu as pltpu
from jax.experimental.pallas import tpu_sc as plsc

N = 10000
E = 320000
HID = 128
HH = HID // 2

NC = 2
NS = 16
NW = NC * NS
CHUNK = 64
CPT = 320
GRP = 64
NGRP = CPT // GRP
EP = NS * CPT * CHUNK
CPWC = 160
NPAD = 10240
RPT = NPAD // NS
CNTW = 16
BN = 1024


def _sc_agg_body(hl_hbm, hh_hbm, srcw_hbm, dstw_hbm, out_hbm,
                 src_g, dst_g, buf0, buf1, hsp, acc_sh, sem0, sem1):
    cid = lax.axis_index("c")
    sid = lax.axis_index("s")
    slab = sid * RPT

    @pl.when(cid == 0)
    def _():
        pltpu.sync_copy(hl_hbm.at[pl.ds(slab, RPT)], hsp.at[pl.ds(slab, RPT)])

    @pl.when(cid == 1)
    def _():
        pltpu.sync_copy(hh_hbm.at[pl.ds(slab, RPT)], hsp.at[pl.ds(slab, RPT)])

    zvec = jnp.zeros((16,), jnp.float32)

    def _zero_row(i, _):
        for j in range(HH // 16):
            buf0[i, pl.ds(j * 16, 16)] = zvec
        return 0

    lax.fori_loop(0, CHUNK, _zero_row, 0)
    for r in range(RPT // CHUNK):
        pltpu.sync_copy(buf0, acc_sh.at[pl.ds(slab + r * CHUNK, CHUNK)])

    plsc.subcore_barrier()

    def _group(g, _):
        pltpu.sync_copy(srcw_hbm.at[sid, pl.ds(g * GRP, GRP)], src_g)
        pltpu.sync_copy(dstw_hbm.at[sid, pl.ds(g * GRP, GRP)], dst_g)

        def _pair(j, _):
            c0 = 2 * j
            c1 = 2 * j + 1
            cp0 = pltpu.async_copy(hsp.at[src_g.at[c0]], buf0, sem0)
            cp1 = pltpu.async_copy(hsp.at[src_g.at[c1]], buf1, sem1)
            cp0.wait()
            pltpu.sync_copy(buf0, acc_sh.at[dst_g.at[c0]], add=True)
            cp1.wait()
            pltpu.sync_copy(buf1, acc_sh.at[dst_g.at[c1]], add=True)
            return 0

        lax.fori_loop(0, GRP // 2, _pair, 0)
        return 0

    lax.fori_loop(0, NGRP, _group, 0)

    plsc.subcore_barrier()

    pltpu.sync_copy(acc_sh.at[pl.ds(slab, RPT)],
                    out_hbm.at[cid, pl.ds(slab, RPT)])


_sc_agg = pl.kernel(
    _sc_agg_body,
    out_type=[jax.ShapeDtypeStruct((NC, NPAD, HH), jnp.float32)],
    mesh=plsc.VectorSubcoreMesh(core_axis_name="c", subcore_axis_name="s"),
    scratch_types=[
        pltpu.VMEM((GRP, CHUNK), jnp.int32),
        pltpu.VMEM((GRP, CHUNK), jnp.int32),
        pltpu.VMEM((CHUNK, HH), jnp.float32),
        pltpu.VMEM((CHUNK, HH), jnp.float32),
        pltpu.VMEM_SHARED((NPAD, HH), jnp.float32),
        pltpu.VMEM_SHARED((NPAD, HH), jnp.float32),
        pltpu.SemaphoreType.DMA,
        pltpu.SemaphoreType.DMA,
    ],
    compiler_params=pltpu.CompilerParams(use_tc_tiling_on_sc=False),
)


def _sc_cnt_body(dstw_hbm, cnt_hbm, dst_v, ones_v, zc_v, cnt_sh):
    cid = lax.axis_index("c")
    sid = lax.axis_index("s")
    wid = sid * NC + cid

    pltpu.sync_copy(dstw_hbm.at[wid], dst_v)

    ovec = jnp.full((16,), 1.0, jnp.float32)
    zvec = jnp.zeros((16,), jnp.float32)

    def _fill(i, _):
        ones_v[i, :] = ovec
        zc_v[i, :] = zvec
        return 0

    lax.fori_loop(0, CHUNK, _fill, 0)
    for r in range(RPT // CHUNK):
        pltpu.sync_copy(zc_v, cnt_sh.at[pl.ds(sid * RPT + r * CHUNK, CHUNK)])

    plsc.subcore_barrier()

    def _chunk(c, _):
        pltpu.sync_copy(ones_v, cnt_sh.at[dst_v.at[c]], add=True)
        return 0

    lax.fori_loop(0, CPWC, _chunk, 0)

    plsc.subcore_barrier()

    pltpu.sync_copy(cnt_sh.at[pl.ds(sid * RPT, RPT)],
                    cnt_hbm.at[cid, pl.ds(sid * RPT, RPT)])


_sc_cnt = pl.kernel(
    _sc_cnt_body,
    out_type=[jax.ShapeDtypeStruct((NC, NPAD, CNTW), jnp.float32)],
    mesh=plsc.VectorSubcoreMesh(core_axis_name="c", subcore_axis_name="s"),
    scratch_types=[
        pltpu.VMEM((CPWC, CHUNK), jnp.int32),
        pltpu.VMEM((CHUNK, CNTW), jnp.float32),
        pltpu.VMEM((CHUNK, CNTW), jnp.float32),
        pltpu.VMEM_SHARED((NPAD, CNTW), jnp.float32),
    ],
    compiler_params=pltpu.CompilerParams(use_tc_tiling_on_sc=False),
)


def _pre_body(x_ref, w_ref, b_ref, ol_ref, oh_ref):
    h = jnp.maximum(
        jnp.dot(x_ref[...], w_ref[...], preferred_element_type=jnp.float32)
        + b_ref[...], 0.0)
    ol_ref[...] = h[:, :HH]
    oh_ref[...] = h[:, HH:]


def _tc_pre(x, w8, b):
    return pl.pallas_call(
        _pre_body,
        grid=(NPAD // BN,),
        in_specs=[
            pl.BlockSpec((BN, 8), lambda i: (i, 0)),
            pl.BlockSpec((8, HID), lambda i: (0, 0)),
            pl.BlockSpec((1, HID), lambda i: (0, 0)),
        ],
        out_specs=[
            pl.BlockSpec((BN, HH), lambda i: (i, 0)),
            pl.BlockSpec((BN, HH), lambda i: (i, 0)),
        ],
        out_shape=[
            jax.ShapeDtypeStruct((NPAD, HH), jnp.float32),
            jax.ShapeDtypeStruct((NPAD, HH), jnp.float32),
        ],
    )(x, w8, b)


def _mid_body(p_ref, c_ref, hl_ref, hh_ref, wl_ref, wr_ref, b_ref,
              ol_ref, oh_ref):
    s = jnp.concatenate([p_ref[0], p_ref[1]], axis=1)
    cnt = c_ref[0, :, 0:1] + c_ref[1, :, 0:1]
    mean = s / jnp.maximum(cnt, 1.0)
    h = jnp.concatenate([hl_ref[...], hh_ref[...]], axis=1)
    o = jnp.maximum(
        jnp.dot(mean, wl_ref[...], preferred_element_type=jnp.float32)
        + jnp.dot(h, wr_ref[...], preferred_element_type=jnp.float32)
        + b_ref[...], 0.0)
    ol_ref[...] = o[:, :HH]
    oh_ref[...] = o[:, HH:]


def _tc_mid(p, c, hl, hh, wlT, wrT, b):
    return pl.pallas_call(
        _mid_body,
        grid=(NPAD // BN,),
        in_specs=[
            pl.BlockSpec((NC, BN, HH), lambda i: (0, i, 0)),
            pl.BlockSpec((NC, BN, CNTW), lambda i: (0, i, 0)),
            pl.BlockSpec((BN, HH), lambda i: (i, 0)),
            pl.BlockSpec((BN, HH), lambda i: (i, 0)),
            pl.BlockSpec((HID, HID), lambda i: (0, 0)),
            pl.BlockSpec((HID, HID), lambda i: (0, 0)),
            pl.BlockSpec((1, HID), lambda i: (0, 0)),
        ],
        out_specs=[
            pl.BlockSpec((BN, HH), lambda i: (i, 0)),
            pl.BlockSpec((BN, HH), lambda i: (i, 0)),
        ],
        out_shape=[
            jax.ShapeDtypeStruct((NPAD, HH), jnp.float32),
            jax.ShapeDtypeStruct((NPAD, HH), jnp.float32),
        ],
    )(p, c, hl, hh, wlT, wrT, b)


def _fin_body(p_ref, c_ref, hl_ref, hh_ref, wl_ref, wr_ref, b_ref,
              wo_ref, bo_ref, o_ref, m_ref):
    i = pl.program_id(0)
    s = jnp.concatenate([p_ref[0], p_ref[1]], axis=1)
    cnt = c_ref[0, :, 0:1] + c_ref[1, :, 0:1]
    mean = s / jnp.maximum(cnt, 1.0)
    h = jnp.concatenate([hl_ref[...], hh_ref[...]], axis=1)
    h2 = jnp.maximum(
        jnp.dot(mean, wl_ref[...], preferred_element_type=jnp.float32)
        + jnp.dot(h, wr_ref[...], preferred_element_type=jnp.float32)
        + b_ref[...], 0.0)
    ns = (jnp.dot(h2, wo_ref[...], preferred_element_type=jnp.float32)
          + bo_ref[...])
    o_ref[...] = ns
    row = i * BN + lax.broadcasted_iota(jnp.int32, (BN, 1), 0)
    valid = (row < N).astype(jnp.float32)
    part = jnp.sum(ns * valid, axis=0, keepdims=True)

    @pl.when(i == 0)
    def _():
        m_ref[...] = jnp.zeros_like(m_ref)

    acc = m_ref[...] + part
    m_ref[...] = jnp.where(i == NPAD // BN - 1,
                           acc * np.float32(1.0 / N), acc)


def _tc_final(p, c, hl, hh, wlT, wrT, b, woT, bo):
    return pl.pallas_call(
        _fin_body,
        grid=(NPAD // BN,),
        in_specs=[
            pl.BlockSpec((NC, BN, HH), lambda i: (0, i, 0)),
            pl.BlockSpec((NC, BN, CNTW), lambda i: (0, i, 0)),
            pl.BlockSpec((BN, HH), lambda i: (i, 0)),
            pl.BlockSpec((BN, HH), lambda i: (i, 0)),
            pl.BlockSpec((HID, HID), lambda i: (0, 0)),
            pl.BlockSpec((HID, HID), lambda i: (0, 0)),
            pl.BlockSpec((1, HID), lambda i: (0, 0)),
            pl.BlockSpec((HID, HID), lambda i: (0, 0)),
            pl.BlockSpec((1, HID), lambda i: (0, 0)),
        ],
        out_specs=[
            pl.BlockSpec((BN, HID), lambda i: (i, 0)),
            pl.BlockSpec((1, HID), lambda i: (0, 0)),
        ],
        out_shape=[
            jax.ShapeDtypeStruct((NPAD, HID), jnp.float32),
            jax.ShapeDtypeStruct((1, HID), jnp.float32),
        ],
    )(p, c, hl, hh, wlT, wrT, b, woT, bo)


def kernel(pos, atomic_number, edge_index,
           W_in, b_in, W1_l, b1, W1_r, W2_l, b2, W2_r, W_out, b_out):
    f32 = jnp.float32
    z = atomic_number.astype(f32)[:, None] / 10.0
    x = jnp.concatenate([z, pos, jnp.zeros((N, 4), f32)], axis=1)
    x = jnp.concatenate([x, jnp.zeros((NPAD - N, 8), f32)], axis=0)
    w8 = jnp.concatenate([W_in.T, jnp.zeros((4, HID), f32)], axis=0)

    src = edge_index[0]
    dst = edge_index[1]
    pad = EP - E
    src_p = jnp.concatenate([src, jnp.zeros((pad,), jnp.int32)])
    dst_p = jnp.concatenate([dst, jnp.full((pad,), N, jnp.int32)])
    srcw = src_p.reshape(NS, CPT, CHUNK)
    dstw = dst_p.reshape(NS, CPT, CHUNK)
    dstwc = dst_p.reshape(NW, CPWC, CHUNK)

    b_in2 = b_in.reshape(1, HID)
    b1_2 = b1.reshape(1, HID)
    b2_2 = b2.reshape(1, HID)
    bo_2 = b_out.reshape(1, HID)

    (cnt,) = _sc_cnt(dstwc)
    h0l, h0h = _tc_pre(x, w8, b_in2)
    (p1,) = _sc_agg(h0l, h0h, srcw, dstw)
    h1l, h1h = _tc_mid(p1, cnt, h0l, h0h, W1_l.T, W1_r.T, b1_2)
    (p2,) = _sc_agg(h1l, h1h, srcw, dstw)
    ns, mvec = _tc_final(p2, cnt, h1l, h1h, W2_l.T, W2_r.T, b2_2, W_out.T, bo_2)
    return mvec.reshape(HID), ns[:N]


# speedup vs baseline: 6.6000x; 1.0148x over previous
import jax
import jax.numpy as jnp
import numpy as np
from jax import lax
from jax.experimental import pallas as pl
from jax.experimental.pallas import tpu as pltpu
from jax.experimental.pallas import tpu_sc as plsc

N = 10000
E = 320000
HID = 128
HH = HID // 2

NC = 2
NS = 16
NW = NC * NS
CHUNK = 128
CPT = 160
GRP = 80
NGRP = CPT // GRP
EP = NS * CPT * CHUNK
CPWC = 80
NPAD = 10240
RPT = NPAD // NS
CNTW = 16
BN = 1024


def _sc_agg_body(hl_hbm, hh_hbm, srcw_hbm, dstw_hbm, out_hbm,
                 src_g, dst_g, buf0, buf1, hsp, acc_sh, sem0, sem1):
    cid = lax.axis_index("c")
    sid = lax.axis_index("s")
    slab = sid * RPT

    @pl.when(cid == 0)
    def _():
        pltpu.sync_copy(hl_hbm.at[pl.ds(slab, RPT)], hsp.at[pl.ds(slab, RPT)])

    @pl.when(cid == 1)
    def _():
        pltpu.sync_copy(hh_hbm.at[pl.ds(slab, RPT)], hsp.at[pl.ds(slab, RPT)])

    zvec = jnp.zeros((16,), jnp.float32)

    def _zero_row(i, _):
        for j in range(HH // 16):
            buf0[i, pl.ds(j * 16, 16)] = zvec
        return 0

    lax.fori_loop(0, CHUNK, _zero_row, 0)
    for r in range(RPT // CHUNK):
        pltpu.sync_copy(buf0, acc_sh.at[pl.ds(slab + r * CHUNK, CHUNK)])

    plsc.subcore_barrier()

    def _group(g, _):
        pltpu.sync_copy(srcw_hbm.at[sid, pl.ds(g * GRP, GRP)], src_g)
        pltpu.sync_copy(dstw_hbm.at[sid, pl.ds(g * GRP, GRP)], dst_g)

        def _pair(j, _):
            c0 = 2 * j
            c1 = 2 * j + 1
            cp0 = pltpu.async_copy(hsp.at[src_g.at[c0]], buf0, sem0)
            cp1 = pltpu.async_copy(hsp.at[src_g.at[c1]], buf1, sem1)
            cp0.wait()
            pltpu.sync_copy(buf0, acc_sh.at[dst_g.at[c0]], add=True)
            cp1.wait()
            pltpu.sync_copy(buf1, acc_sh.at[dst_g.at[c1]], add=True)
            return 0

        lax.fori_loop(0, GRP // 2, _pair, 0)
        return 0

    lax.fori_loop(0, NGRP, _group, 0)

    plsc.subcore_barrier()

    pltpu.sync_copy(acc_sh.at[pl.ds(slab, RPT)],
                    out_hbm.at[cid, pl.ds(slab, RPT)])


_sc_agg = pl.kernel(
    _sc_agg_body,
    out_type=[jax.ShapeDtypeStruct((NC, NPAD, HH), jnp.float32)],
    mesh=plsc.VectorSubcoreMesh(core_axis_name="c", subcore_axis_name="s"),
    scratch_types=[
        pltpu.VMEM((GRP, CHUNK), jnp.int32),
        pltpu.VMEM((GRP, CHUNK), jnp.int32),
        pltpu.VMEM((CHUNK, HH), jnp.float32),
        pltpu.VMEM((CHUNK, HH), jnp.float32),
        pltpu.VMEM_SHARED((NPAD, HH), jnp.float32),
        pltpu.VMEM_SHARED((NPAD, HH), jnp.float32),
        pltpu.SemaphoreType.DMA,
        pltpu.SemaphoreType.DMA,
    ],
    compiler_params=pltpu.CompilerParams(use_tc_tiling_on_sc=False),
)


def _sc_cnt_body(dstw_hbm, cnt_hbm, dst_v, ones_v, zc_v, cnt_sh):
    cid = lax.axis_index("c")
    sid = lax.axis_index("s")
    wid = sid * NC + cid

    pltpu.sync_copy(dstw_hbm.at[wid], dst_v)

    ovec = jnp.full((16,), 1.0, jnp.float32)
    zvec = jnp.zeros((16,), jnp.float32)

    def _fill(i, _):
        ones_v[i, :] = ovec
        zc_v[i, :] = zvec
        return 0

    lax.fori_loop(0, CHUNK, _fill, 0)
    for r in range(RPT // CHUNK):
        pltpu.sync_copy(zc_v, cnt_sh.at[pl.ds(sid * RPT + r * CHUNK, CHUNK)])

    plsc.subcore_barrier()

    def _chunk(c, _):
        pltpu.sync_copy(ones_v, cnt_sh.at[dst_v.at[c]], add=True)
        return 0

    lax.fori_loop(0, CPWC, _chunk, 0)

    plsc.subcore_barrier()

    pltpu.sync_copy(cnt_sh.at[pl.ds(sid * RPT, RPT)],
                    cnt_hbm.at[cid, pl.ds(sid * RPT, RPT)])


_sc_cnt = pl.kernel(
    _sc_cnt_body,
    out_type=[jax.ShapeDtypeStruct((NC, NPAD, CNTW), jnp.float32)],
    mesh=plsc.VectorSubcoreMesh(core_axis_name="c", subcore_axis_name="s"),
    scratch_types=[
        pltpu.VMEM((CPWC, CHUNK), jnp.int32),
        pltpu.VMEM((CHUNK, CNTW), jnp.float32),
        pltpu.VMEM((CHUNK, CNTW), jnp.float32),
        pltpu.VMEM_SHARED((NPAD, CNTW), jnp.float32),
    ],
    compiler_params=pltpu.CompilerParams(use_tc_tiling_on_sc=False),
)


def _pre_body(x_ref, w_ref, b_ref, ol_ref, oh_ref):
    h = jnp.maximum(
        jnp.dot(x_ref[...], w_ref[...], preferred_element_type=jnp.float32)
        + b_ref[...], 0.0)
    ol_ref[...] = h[:, :HH]
    oh_ref[...] = h[:, HH:]


def _tc_pre(x, w8, b):
    return pl.pallas_call(
        _pre_body,
        grid=(NPAD // BN,),
        in_specs=[
            pl.BlockSpec((BN, 8), lambda i: (i, 0)),
            pl.BlockSpec((8, HID), lambda i: (0, 0)),
            pl.BlockSpec((1, HID), lambda i: (0, 0)),
        ],
        out_specs=[
            pl.BlockSpec((BN, HH), lambda i: (i, 0)),
            pl.BlockSpec((BN, HH), lambda i: (i, 0)),
        ],
        out_shape=[
            jax.ShapeDtypeStruct((NPAD, HH), jnp.float32),
            jax.ShapeDtypeStruct((NPAD, HH), jnp.float32),
        ],
    )(x, w8, b)


def _mid_body(p_ref, c_ref, hl_ref, hh_ref, wl_ref, wr_ref, b_ref,
              ol_ref, oh_ref):
    s = jnp.concatenate([p_ref[0], p_ref[1]], axis=1)
    cnt = c_ref[0, :, 0:1] + c_ref[1, :, 0:1]
    mean = s / jnp.maximum(cnt, 1.0)
    h = jnp.concatenate([hl_ref[...], hh_ref[...]], axis=1)
    o = jnp.maximum(
        jnp.dot(mean, wl_ref[...], preferred_element_type=jnp.float32)
        + jnp.dot(h, wr_ref[...], preferred_element_type=jnp.float32)
        + b_ref[...], 0.0)
    ol_ref[...] = o[:, :HH]
    oh_ref[...] = o[:, HH:]


def _tc_mid(p, c, hl, hh, wlT, wrT, b):
    return pl.pallas_call(
        _mid_body,
        grid=(NPAD // BN,),
        in_specs=[
            pl.BlockSpec((NC, BN, HH), lambda i: (0, i, 0)),
            pl.BlockSpec((NC, BN, CNTW), lambda i: (0, i, 0)),
            pl.BlockSpec((BN, HH), lambda i: (i, 0)),
            pl.BlockSpec((BN, HH), lambda i: (i, 0)),
            pl.BlockSpec((HID, HID), lambda i: (0, 0)),
            pl.BlockSpec((HID, HID), lambda i: (0, 0)),
            pl.BlockSpec((1, HID), lambda i: (0, 0)),
        ],
        out_specs=[
            pl.BlockSpec((BN, HH), lambda i: (i, 0)),
            pl.BlockSpec((BN, HH), lambda i: (i, 0)),
        ],
        out_shape=[
            jax.ShapeDtypeStruct((NPAD, HH), jnp.float32),
            jax.ShapeDtypeStruct((NPAD, HH), jnp.float32),
        ],
    )(p, c, hl, hh, wlT, wrT, b)


def _fin_body(p_ref, c_ref, hl_ref, hh_ref, wl_ref, wr_ref, b_ref,
              wo_ref, bo_ref, o_ref, m_ref):
    i = pl.program_id(0)
    s = jnp.concatenate([p_ref[0], p_ref[1]], axis=1)
    cnt = c_ref[0, :, 0:1] + c_ref[1, :, 0:1]
    mean = s / jnp.maximum(cnt, 1.0)
    h = jnp.concatenate([hl_ref[...], hh_ref[...]], axis=1)
    h2 = jnp.maximum(
        jnp.dot(mean, wl_ref[...], preferred_element_type=jnp.float32)
        + jnp.dot(h, wr_ref[...], preferred_element_type=jnp.float32)
        + b_ref[...], 0.0)
    ns = (jnp.dot(h2, wo_ref[...], preferred_element_type=jnp.float32)
          + bo_ref[...])
    o_ref[...] = ns
    row = i * BN + lax.broadcasted_iota(jnp.int32, (BN, 1), 0)
    valid = (row < N).astype(jnp.float32)
    part = jnp.sum(ns * valid, axis=0, keepdims=True)

    @pl.when(i == 0)
    def _():
        m_ref[...] = jnp.zeros_like(m_ref)

    acc = m_ref[...] + part
    m_ref[...] = jnp.where(i == NPAD // BN - 1,
                           acc * np.float32(1.0 / N), acc)


def _tc_final(p, c, hl, hh, wlT, wrT, b, woT, bo):
    return pl.pallas_call(
        _fin_body,
        grid=(NPAD // BN,),
        in_specs=[
            pl.BlockSpec((NC, BN, HH), lambda i: (0, i, 0)),
            pl.BlockSpec((NC, BN, CNTW), lambda i: (0, i, 0)),
            pl.BlockSpec((BN, HH), lambda i: (i, 0)),
            pl.BlockSpec((BN, HH), lambda i: (i, 0)),
            pl.BlockSpec((HID, HID), lambda i: (0, 0)),
            pl.BlockSpec((HID, HID), lambda i: (0, 0)),
            pl.BlockSpec((1, HID), lambda i: (0, 0)),
            pl.BlockSpec((HID, HID), lambda i: (0, 0)),
            pl.BlockSpec((1, HID), lambda i: (0, 0)),
        ],
        out_specs=[
            pl.BlockSpec((BN, HID), lambda i: (i, 0)),
            pl.BlockSpec((1, HID), lambda i: (0, 0)),
        ],
        out_shape=[
            jax.ShapeDtypeStruct((NPAD, HID), jnp.float32),
            jax.ShapeDtypeStruct((1, HID), jnp.float32),
        ],
    )(p, c, hl, hh, wlT, wrT, b, woT, bo)


def kernel(pos, atomic_number, edge_index,
           W_in, b_in, W1_l, b1, W1_r, W2_l, b2, W2_r, W_out, b_out):
    f32 = jnp.float32
    z = atomic_number.astype(f32)[:, None] / 10.0
    x = jnp.concatenate([z, pos, jnp.zeros((N, 4), f32)], axis=1)
    x = jnp.concatenate([x, jnp.zeros((NPAD - N, 8), f32)], axis=0)
    w8 = jnp.concatenate([W_in.T, jnp.zeros((4, HID), f32)], axis=0)

    src = edge_index[0]
    dst = edge_index[1]
    pad = EP - E
    src_p = jnp.concatenate([src, jnp.zeros((pad,), jnp.int32)])
    dst_p = jnp.concatenate([dst, jnp.full((pad,), N, jnp.int32)])
    srcw = src_p.reshape(NS, CPT, CHUNK)
    dstw = dst_p.reshape(NS, CPT, CHUNK)
    dstwc = dst_p.reshape(NW, CPWC, CHUNK)

    b_in2 = b_in.reshape(1, HID)
    b1_2 = b1.reshape(1, HID)
    b2_2 = b2.reshape(1, HID)
    bo_2 = b_out.reshape(1, HID)

    (cnt,) = _sc_cnt(dstwc)
    h0l, h0h = _tc_pre(x, w8, b_in2)
    (p1,) = _sc_agg(h0l, h0h, srcw, dstw)
    h1l, h1h = _tc_mid(p1, cnt, h0l, h0h, W1_l.T, W1_r.T, b1_2)
    (p2,) = _sc_agg(h1l, h1h, srcw, dstw)
    ns, mvec = _tc_final(p2, cnt, h1l, h1h, W2_l.T, W2_r.T, b2_2, W_out.T, bo_2)
    return mvec.reshape(HID), ns[:N]


# speedup vs baseline: 7.6299x; 1.1560x over previous
import jax
import jax.numpy as jnp
import numpy as np
from jax import lax
from jax.experimental import pallas as pl
from jax.experimental.pallas import tpu as pltpu
from jax.experimental.pallas import tpu_sc as plsc

N = 10000
E = 320000
HID = 128
HH = HID // 2

NC = 2
NS = 16
NW = NC * NS
CHUNK = 128
CPT = 160
GRP = 40
NBUF = 4
NGRP = CPT // GRP
EP = NS * CPT * CHUNK
CPWC = 80
NPAD = 10240
RPT = NPAD // NS
CNTW = 16
BN = 1024


def _sc_agg_body(hl_hbm, hh_hbm, srcw_hbm, dstw_hbm, out_hbm,
                 src_g, dst_g, bufs, hsp, acc_sh, gsems, ssems):
    cid = lax.axis_index("c")
    sid = lax.axis_index("s")
    slab = sid * RPT
    buf0 = bufs[0]

    @pl.when(cid == 0)
    def _():
        pltpu.sync_copy(hl_hbm.at[pl.ds(slab, RPT)], hsp.at[pl.ds(slab, RPT)])

    @pl.when(cid == 1)
    def _():
        pltpu.sync_copy(hh_hbm.at[pl.ds(slab, RPT)], hsp.at[pl.ds(slab, RPT)])

    zvec = jnp.zeros((16,), jnp.float32)

    def _zero_row(i, _):
        for j in range(HH // 16):
            buf0[i, pl.ds(j * 16, 16)] = zvec
        return 0

    lax.fori_loop(0, CHUNK, _zero_row, 0)
    for r in range(RPT // CHUNK):
        pltpu.sync_copy(buf0, acc_sh.at[pl.ds(slab + r * CHUNK, CHUNK)])

    plsc.subcore_barrier()

    def _group(g, _):
        @pl.when(g > 0)
        def _():
            for i in range(NBUF):
                pltpu.make_async_copy(
                    hl_hbm.at[pl.ds(0, CHUNK)], bufs[i], ssems[i]).wait()

        pltpu.sync_copy(srcw_hbm.at[sid, pl.ds(g * GRP, GRP)], src_g)
        pltpu.sync_copy(dstw_hbm.at[sid, pl.ds(g * GRP, GRP)], dst_g)

        def _quad(k, _):
            base = NBUF * k
            cps = []
            for i in range(NBUF):
                @pl.when(k > 0)
                def _(i=i):
                    pltpu.make_async_copy(
                        hl_hbm.at[pl.ds(0, CHUNK)], bufs[i], ssems[i]).wait()
                cps.append(pltpu.async_copy(
                    hsp.at[src_g.at[base + i]], bufs[i], gsems[i]))
            for i in range(NBUF):
                cps[i].wait()
                pltpu.async_copy(bufs[i], acc_sh.at[dst_g.at[base + i]],
                                 ssems[i], add=True)
            return 0

        lax.fori_loop(0, GRP // NBUF, _quad, 0)
        return 0

    lax.fori_loop(0, NGRP, _group, 0)

    for i in range(NBUF):
        pltpu.make_async_copy(
            hl_hbm.at[pl.ds(0, CHUNK)], bufs[i], ssems[i]).wait()

    plsc.subcore_barrier()

    pltpu.sync_copy(acc_sh.at[pl.ds(slab, RPT)],
                    out_hbm.at[cid, pl.ds(slab, RPT)])


_sc_agg = pl.kernel(
    _sc_agg_body,
    out_type=[jax.ShapeDtypeStruct((NC, NPAD, HH), jnp.float32)],
    mesh=plsc.VectorSubcoreMesh(core_axis_name="c", subcore_axis_name="s"),
    scratch_types=[
        pltpu.VMEM((GRP, CHUNK), jnp.int32),
        pltpu.VMEM((GRP, CHUNK), jnp.int32),
        [pltpu.VMEM((CHUNK, HH), jnp.float32) for _ in range(NBUF)],
        pltpu.VMEM_SHARED((NPAD, HH), jnp.float32),
        pltpu.VMEM_SHARED((NPAD, HH), jnp.float32),
        [pltpu.SemaphoreType.DMA for _ in range(NBUF)],
        [pltpu.SemaphoreType.DMA for _ in range(NBUF)],
    ],
    compiler_params=pltpu.CompilerParams(use_tc_tiling_on_sc=False),
)


def _sc_cnt_body(dstw_hbm, cnt_hbm, dst_v, ones_v, zc_v, cnt_sh):
    cid = lax.axis_index("c")
    sid = lax.axis_index("s")
    wid = sid * NC + cid

    pltpu.sync_copy(dstw_hbm.at[wid], dst_v)

    ovec = jnp.full((16,), 1.0, jnp.float32)
    zvec = jnp.zeros((16,), jnp.float32)

    def _fill(i, _):
        ones_v[i, :] = ovec
        zc_v[i, :] = zvec
        return 0

    lax.fori_loop(0, CHUNK, _fill, 0)
    for r in range(RPT // CHUNK):
        pltpu.sync_copy(zc_v, cnt_sh.at[pl.ds(sid * RPT + r * CHUNK, CHUNK)])

    plsc.subcore_barrier()

    def _chunk(c, _):
        pltpu.sync_copy(ones_v, cnt_sh.at[dst_v.at[c]], add=True)
        return 0

    lax.fori_loop(0, CPWC, _chunk, 0)

    plsc.subcore_barrier()

    pltpu.sync_copy(cnt_sh.at[pl.ds(sid * RPT, RPT)],
                    cnt_hbm.at[cid, pl.ds(sid * RPT, RPT)])


_sc_cnt = pl.kernel(
    _sc_cnt_body,
    out_type=[jax.ShapeDtypeStruct((NC, NPAD, CNTW), jnp.float32)],
    mesh=plsc.VectorSubcoreMesh(core_axis_name="c", subcore_axis_name="s"),
    scratch_types=[
        pltpu.VMEM((CPWC, CHUNK), jnp.int32),
        pltpu.VMEM((CHUNK, CNTW), jnp.float32),
        pltpu.VMEM((CHUNK, CNTW), jnp.float32),
        pltpu.VMEM_SHARED((NPAD, CNTW), jnp.float32),
    ],
    compiler_params=pltpu.CompilerParams(use_tc_tiling_on_sc=False),
)


def _pre_body(x_ref, w_ref, b_ref, ol_ref, oh_ref):
    h = jnp.maximum(
        jnp.dot(x_ref[...], w_ref[...], preferred_element_type=jnp.float32)
        + b_ref[...], 0.0)
    ol_ref[...] = h[:, :HH]
    oh_ref[...] = h[:, HH:]


def _tc_pre(x, w8, b):
    return pl.pallas_call(
        _pre_body,
        grid=(NPAD // BN,),
        in_specs=[
            pl.BlockSpec((BN, 8), lambda i: (i, 0)),
            pl.BlockSpec((8, HID), lambda i: (0, 0)),
            pl.BlockSpec((1, HID), lambda i: (0, 0)),
        ],
        out_specs=[
            pl.BlockSpec((BN, HH), lambda i: (i, 0)),
            pl.BlockSpec((BN, HH), lambda i: (i, 0)),
        ],
        out_shape=[
            jax.ShapeDtypeStruct((NPAD, HH), jnp.float32),
            jax.ShapeDtypeStruct((NPAD, HH), jnp.float32),
        ],
    )(x, w8, b)


def _mid_body(p_ref, c_ref, hl_ref, hh_ref, wl_ref, wr_ref, b_ref,
              ol_ref, oh_ref):
    s = jnp.concatenate([p_ref[0], p_ref[1]], axis=1)
    cnt = c_ref[0, :, 0:1] + c_ref[1, :, 0:1]
    mean = s / jnp.maximum(cnt, 1.0)
    h = jnp.concatenate([hl_ref[...], hh_ref[...]], axis=1)
    o = jnp.maximum(
        jnp.dot(mean, wl_ref[...], preferred_element_type=jnp.float32)
        + jnp.dot(h, wr_ref[...], preferred_element_type=jnp.float32)
        + b_ref[...], 0.0)
    ol_ref[...] = o[:, :HH]
    oh_ref[...] = o[:, HH:]


def _tc_mid(p, c, hl, hh, wlT, wrT, b):
    return pl.pallas_call(
        _mid_body,
        grid=(NPAD // BN,),
        in_specs=[
            pl.BlockSpec((NC, BN, HH), lambda i: (0, i, 0)),
            pl.BlockSpec((NC, BN, CNTW), lambda i: (0, i, 0)),
            pl.BlockSpec((BN, HH), lambda i: (i, 0)),
            pl.BlockSpec((BN, HH), lambda i: (i, 0)),
            pl.BlockSpec((HID, HID), lambda i: (0, 0)),
            pl.BlockSpec((HID, HID), lambda i: (0, 0)),
            pl.BlockSpec((1, HID), lambda i: (0, 0)),
        ],
        out_specs=[
            pl.BlockSpec((BN, HH), lambda i: (i, 0)),
            pl.BlockSpec((BN, HH), lambda i: (i, 0)),
        ],
        out_shape=[
            jax.ShapeDtypeStruct((NPAD, HH), jnp.float32),
            jax.ShapeDtypeStruct((NPAD, HH), jnp.float32),
        ],
    )(p, c, hl, hh, wlT, wrT, b)


def _fin_body(p_ref, c_ref, hl_ref, hh_ref, wl_ref, wr_ref, b_ref,
              wo_ref, bo_ref, o_ref, m_ref):
    i = pl.program_id(0)
    s = jnp.concatenate([p_ref[0], p_ref[1]], axis=1)
    cnt = c_ref[0, :, 0:1] + c_ref[1, :, 0:1]
    mean = s / jnp.maximum(cnt, 1.0)
    h = jnp.concatenate([hl_ref[...], hh_ref[...]], axis=1)
    h2 = jnp.maximum(
        jnp.dot(mean, wl_ref[...], preferred_element_type=jnp.float32)
        + jnp.dot(h, wr_ref[...], preferred_element_type=jnp.float32)
        + b_ref[...], 0.0)
    ns = (jnp.dot(h2, wo_ref[...], preferred_element_type=jnp.float32)
          + bo_ref[...])
    o_ref[...] = ns
    row = i * BN + lax.broadcasted_iota(jnp.int32, (BN, 1), 0)
    valid = (row < N).astype(jnp.float32)
    part = jnp.sum(ns * valid, axis=0, keepdims=True)

    @pl.when(i == 0)
    def _():
        m_ref[...] = jnp.zeros_like(m_ref)

    acc = m_ref[...] + part
    m_ref[...] = jnp.where(i == NPAD // BN - 1,
                           acc * np.float32(1.0 / N), acc)


def _tc_final(p, c, hl, hh, wlT, wrT, b, woT, bo):
    return pl.pallas_call(
        _fin_body,
        grid=(NPAD // BN,),
        in_specs=[
            pl.BlockSpec((NC, BN, HH), lambda i: (0, i, 0)),
            pl.BlockSpec((NC, BN, CNTW), lambda i: (0, i, 0)),
            pl.BlockSpec((BN, HH), lambda i: (i, 0)),
            pl.BlockSpec((BN, HH), lambda i: (i, 0)),
            pl.BlockSpec((HID, HID), lambda i: (0, 0)),
            pl.BlockSpec((HID, HID), lambda i: (0, 0)),
            pl.BlockSpec((1, HID), lambda i: (0, 0)),
            pl.BlockSpec((HID, HID), lambda i: (0, 0)),
            pl.BlockSpec((1, HID), lambda i: (0, 0)),
        ],
        out_specs=[
            pl.BlockSpec((BN, HID), lambda i: (i, 0)),
            pl.BlockSpec((1, HID), lambda i: (0, 0)),
        ],
        out_shape=[
            jax.ShapeDtypeStruct((NPAD, HID), jnp.float32),
            jax.ShapeDtypeStruct((1, HID), jnp.float32),
        ],
    )(p, c, hl, hh, wlT, wrT, b, woT, bo)


def kernel(pos, atomic_number, edge_index,
           W_in, b_in, W1_l, b1, W1_r, W2_l, b2, W2_r, W_out, b_out):
    f32 = jnp.float32
    z = atomic_number.astype(f32)[:, None] / 10.0
    x = jnp.concatenate([z, pos, jnp.zeros((N, 4), f32)], axis=1)
    x = jnp.concatenate([x, jnp.zeros((NPAD - N, 8), f32)], axis=0)
    w8 = jnp.concatenate([W_in.T, jnp.zeros((4, HID), f32)], axis=0)

    src = edge_index[0]
    dst = edge_index[1]
    pad = EP - E
    src_p = jnp.concatenate([src, jnp.zeros((pad,), jnp.int32)])
    dst_p = jnp.concatenate([dst, jnp.full((pad,), N, jnp.int32)])
    srcw = src_p.reshape(NS, CPT, CHUNK)
    dstw = dst_p.reshape(NS, CPT, CHUNK)
    dstwc = dst_p.reshape(NW, CPWC, CHUNK)

    b_in2 = b_in.reshape(1, HID)
    b1_2 = b1.reshape(1, HID)
    b2_2 = b2.reshape(1, HID)
    bo_2 = b_out.reshape(1, HID)

    (cnt,) = _sc_cnt(dstwc)
    h0l, h0h = _tc_pre(x, w8, b_in2)
    (p1,) = _sc_agg(h0l, h0h, srcw, dstw)
    h1l, h1h = _tc_mid(p1, cnt, h0l, h0h, W1_l.T, W1_r.T, b1_2)
    (p2,) = _sc_agg(h1l, h1h, srcw, dstw)
    ns, mvec = _tc_final(p2, cnt, h1l, h1h, W2_l.T, W2_r.T, b2_2, W_out.T, bo_2)
    return mvec.reshape(HID), ns[:N]


# speedup vs baseline: 8.8877x; 1.1648x over previous
import jax
import jax.numpy as jnp
import numpy as np
from jax import lax
from jax.experimental import pallas as pl
from jax.experimental.pallas import tpu as pltpu
from jax.experimental.pallas import tpu_sc as plsc

N = 10000
E = 320000
HID = 128
HH = HID // 2

NC = 2
NS = 16
NW = NC * NS
CHUNK = 128
CPT = 160
GRP = 20
NBUF = 5
NGRP = CPT // GRP
EP = NS * CPT * CHUNK
CPWC = 80
NPAD = 10240
RPT = NPAD // NS
CNTW = 16
BN = 1024


def _sc_agg_body(h_hbm, ei_hbm, out_hbm,
                 src_g, dst_g, bufs, hsp, acc_sh, gsems, ssems):
    cid = lax.axis_index("c")
    sid = lax.axis_index("s")
    slab = sid * RPT
    buf0 = bufs[0]

    @pl.when(cid == 0)
    def _():
        pltpu.sync_copy(h_hbm.at[pl.ds(slab, RPT), pl.ds(0, HH)],
                        hsp.at[pl.ds(slab, RPT)])

    @pl.when(cid == 1)
    def _():
        pltpu.sync_copy(h_hbm.at[pl.ds(slab, RPT), pl.ds(HH, HH)],
                        hsp.at[pl.ds(slab, RPT)])

    zvec = jnp.zeros((16,), jnp.float32)

    def _zero_row(i, _):
        for j in range(HH // 16):
            buf0[i, pl.ds(j * 16, 16)] = zvec
        return 0

    lax.fori_loop(0, CHUNK, _zero_row, 0)
    for r in range(RPT // CHUNK):
        pltpu.sync_copy(buf0, acc_sh.at[pl.ds(slab + r * CHUNK, CHUNK)])

    plsc.subcore_barrier()

    def _drain(i):
        pltpu.make_async_copy(
            h_hbm.at[pl.ds(0, CHUNK), pl.ds(0, HH)], bufs[i], ssems[i]).wait()

    def _group(g, _):
        @pl.when(g > 0)
        def _():
            for i in range(NBUF):
                _drain(i)

        pltpu.sync_copy(ei_hbm.at[0, sid, pl.ds(g * GRP, GRP)], src_g)
        pltpu.sync_copy(ei_hbm.at[1, sid, pl.ds(g * GRP, GRP)], dst_g)

        def _quad(k, _):
            base = NBUF * k
            cps = []
            for i in range(NBUF):
                @pl.when(k > 0)
                def _(i=i):
                    _drain(i)
                cps.append(pltpu.async_copy(
                    hsp.at[src_g.at[base + i]], bufs[i], gsems[i]))
            for i in range(NBUF):
                cps[i].wait()
                pltpu.async_copy(bufs[i], acc_sh.at[dst_g.at[base + i]],
                                 ssems[i], add=True)
            return 0

        lax.fori_loop(0, GRP // NBUF, _quad, 0)
        return 0

    lax.fori_loop(0, NGRP, _group, 0)

    for i in range(NBUF):
        _drain(i)

    plsc.subcore_barrier()

    @pl.when(cid == 0)
    def _():
        pltpu.sync_copy(acc_sh.at[pl.ds(slab, RPT)],
                        out_hbm.at[pl.ds(slab, RPT), pl.ds(0, HH)])

    @pl.when(cid == 1)
    def _():
        pltpu.sync_copy(acc_sh.at[pl.ds(slab, RPT)],
                        out_hbm.at[pl.ds(slab, RPT), pl.ds(HH, HH)])


_sc_agg = pl.kernel(
    _sc_agg_body,
    out_type=[jax.ShapeDtypeStruct((NPAD, HID), jnp.float32)],
    mesh=plsc.VectorSubcoreMesh(core_axis_name="c", subcore_axis_name="s"),
    scratch_types=[
        pltpu.VMEM((GRP, CHUNK), jnp.int32),
        pltpu.VMEM((GRP, CHUNK), jnp.int32),
        [pltpu.VMEM((CHUNK, HH), jnp.float32) for _ in range(NBUF)],
        pltpu.VMEM_SHARED((NPAD, HH), jnp.float32),
        pltpu.VMEM_SHARED((NPAD, HH), jnp.float32),
        [pltpu.SemaphoreType.DMA for _ in range(NBUF)],
        [pltpu.SemaphoreType.DMA for _ in range(NBUF)],
    ],
    compiler_params=pltpu.CompilerParams(use_tc_tiling_on_sc=False),
)


def _sc_cnt_body(ei_hbm, cnt_hbm, dst_v, ones_v, zc_v, cnt_sh):
    cid = lax.axis_index("c")
    sid = lax.axis_index("s")
    wid = sid * NC + cid

    pltpu.sync_copy(ei_hbm.at[1, wid], dst_v)

    ovec = jnp.full((16,), 1.0, jnp.float32)
    zvec = jnp.zeros((16,), jnp.float32)

    def _fill(i, _):
        ones_v[i, :] = ovec
        zc_v[i, :] = zvec
        return 0

    lax.fori_loop(0, CHUNK, _fill, 0)
    for r in range(RPT // CHUNK):
        pltpu.sync_copy(zc_v, cnt_sh.at[pl.ds(sid * RPT + r * CHUNK, CHUNK)])

    plsc.subcore_barrier()

    def _chunk(c, _):
        pltpu.sync_copy(ones_v, cnt_sh.at[dst_v.at[c]], add=True)
        return 0

    lax.fori_loop(0, CPWC, _chunk, 0)

    plsc.subcore_barrier()

    pltpu.sync_copy(cnt_sh.at[pl.ds(sid * RPT, RPT)],
                    cnt_hbm.at[cid, pl.ds(sid * RPT, RPT)])


_sc_cnt = pl.kernel(
    _sc_cnt_body,
    out_type=[jax.ShapeDtypeStruct((NC, NPAD, CNTW), jnp.float32)],
    mesh=plsc.VectorSubcoreMesh(core_axis_name="c", subcore_axis_name="s"),
    scratch_types=[
        pltpu.VMEM((CPWC, CHUNK), jnp.int32),
        pltpu.VMEM((CHUNK, CNTW), jnp.float32),
        pltpu.VMEM((CHUNK, CNTW), jnp.float32),
        pltpu.VMEM_SHARED((NPAD, CNTW), jnp.float32),
    ],
    compiler_params=pltpu.CompilerParams(use_tc_tiling_on_sc=False),
)


def _pre_body(x_ref, w_ref, b_ref, o_ref):
    o_ref[...] = jnp.maximum(
        jnp.dot(x_ref[...], w_ref[...], preferred_element_type=jnp.float32)
        + b_ref[...], 0.0)


def _tc_pre(x, w8, b):
    return pl.pallas_call(
        _pre_body,
        grid=(NPAD // BN,),
        in_specs=[
            pl.BlockSpec((BN, 8), lambda i: (i, 0)),
            pl.BlockSpec((8, HID), lambda i: (0, 0)),
            pl.BlockSpec((1, HID), lambda i: (0, 0)),
        ],
        out_specs=pl.BlockSpec((BN, HID), lambda i: (i, 0)),
        out_shape=jax.ShapeDtypeStruct((NPAD, HID), jnp.float32),
    )(x, w8, b)


def _mid_body(p_ref, c_ref, h_ref, wl_ref, wr_ref, b_ref, o_ref):
    cnt = c_ref[0, :, 0:1] + c_ref[1, :, 0:1]
    mean = p_ref[...] / jnp.maximum(cnt, 1.0)
    o_ref[...] = jnp.maximum(
        jnp.dot(mean, wl_ref[...], preferred_element_type=jnp.float32)
        + jnp.dot(h_ref[...], wr_ref[...], preferred_element_type=jnp.float32)
        + b_ref[...], 0.0)


def _tc_mid(p, c, h, wlT, wrT, b):
    return pl.pallas_call(
        _mid_body,
        grid=(NPAD // BN,),
        in_specs=[
            pl.BlockSpec((BN, HID), lambda i: (i, 0)),
            pl.BlockSpec((NC, BN, CNTW), lambda i: (0, i, 0)),
            pl.BlockSpec((BN, HID), lambda i: (i, 0)),
            pl.BlockSpec((HID, HID), lambda i: (0, 0)),
            pl.BlockSpec((HID, HID), lambda i: (0, 0)),
            pl.BlockSpec((1, HID), lambda i: (0, 0)),
        ],
        out_specs=pl.BlockSpec((BN, HID), lambda i: (i, 0)),
        out_shape=jax.ShapeDtypeStruct((NPAD, HID), jnp.float32),
    )(p, c, h, wlT, wrT, b)


BNF = 1000


def _fin_body(p_ref, c_ref, h_ref, wl_ref, wr_ref, b_ref,
              wo_ref, bo_ref, o_ref, m_ref):
    i = pl.program_id(0)
    cnt = c_ref[0, :, 0:1] + c_ref[1, :, 0:1]
    mean = p_ref[...] / jnp.maximum(cnt, 1.0)
    h2 = jnp.maximum(
        jnp.dot(mean, wl_ref[...], preferred_element_type=jnp.float32)
        + jnp.dot(h_ref[...], wr_ref[...], preferred_element_type=jnp.float32)
        + b_ref[...], 0.0)
    ns = (jnp.dot(h2, wo_ref[...], preferred_element_type=jnp.float32)
          + bo_ref[...])
    o_ref[...] = ns
    part = jnp.sum(ns, axis=0, keepdims=True)

    @pl.when(i == 0)
    def _():
        m_ref[...] = jnp.zeros_like(m_ref)

    acc = m_ref[...] + part
    m_ref[...] = jnp.where(i == N // BNF - 1,
                           acc * np.float32(1.0 / N), acc)


def _tc_final(p, c, h, wlT, wrT, b, woT, bo):
    return pl.pallas_call(
        _fin_body,
        grid=(N // BNF,),
        in_specs=[
            pl.BlockSpec((BNF, HID), lambda i: (i, 0)),
            pl.BlockSpec((NC, BNF, CNTW), lambda i: (0, i, 0)),
            pl.BlockSpec((BNF, HID), lambda i: (i, 0)),
            pl.BlockSpec((HID, HID), lambda i: (0, 0)),
            pl.BlockSpec((HID, HID), lambda i: (0, 0)),
            pl.BlockSpec((1, HID), lambda i: (0, 0)),
            pl.BlockSpec((HID, HID), lambda i: (0, 0)),
            pl.BlockSpec((1, HID), lambda i: (0, 0)),
        ],
        out_specs=[
            pl.BlockSpec((BNF, HID), lambda i: (i, 0)),
            pl.BlockSpec((1, HID), lambda i: (0, 0)),
        ],
        out_shape=[
            jax.ShapeDtypeStruct((N, HID), jnp.float32),
            jax.ShapeDtypeStruct((1, HID), jnp.float32),
        ],
    )(p, c, h, wlT, wrT, b, woT, bo)


def kernel(pos, atomic_number, edge_index,
           W_in, b_in, W1_l, b1, W1_r, W2_l, b2, W2_r, W_out, b_out):
    f32 = jnp.float32
    z = atomic_number.astype(f32)[:, None] / 10.0
    x = jnp.concatenate([z, pos, jnp.zeros((N, 4), f32)], axis=1)
    x = jnp.concatenate([x, jnp.zeros((NPAD - N, 8), f32)], axis=0)
    w8 = jnp.concatenate([W_in.T, jnp.zeros((4, HID), f32)], axis=0)

    ei_p = jnp.pad(edge_index, ((0, 0), (0, EP - E)), constant_values=N)
    ei_w = ei_p.reshape(2, NS, CPT, CHUNK)
    ei_c = ei_p.reshape(2, NW, CPWC, CHUNK)

    b_in2 = b_in.reshape(1, HID)
    b1_2 = b1.reshape(1, HID)
    b2_2 = b2.reshape(1, HID)
    bo_2 = b_out.reshape(1, HID)

    (cnt,) = _sc_cnt(ei_c)
    h0 = _tc_pre(x, w8, b_in2)
    (p1,) = _sc_agg(h0, ei_w)
    h1 = _tc_mid(p1, cnt, h0, W1_l.T, W1_r.T, b1_2)
    (p2,) = _sc_agg(h1, ei_w)
    ns, mvec = _tc_final(p2, cnt, h1, W2_l.T, W2_r.T, b2_2, W_out.T, bo_2)
    return mvec.reshape(HID), ns
